# Initial kernel scaffold; baseline (speedup 1.0000x reference)
import jax
import jax.numpy as jnp
from jax.experimental import pallas as pl


def kernel(outputs, targets):
    raise NotImplementedError("write your pallas kernel here")


# speedup vs baseline: 101.0430x; 101.0430x over previous
import functools

import jax
import jax.numpy as jnp
import numpy as np
from jax.experimental import pallas as pl
from jax.experimental.pallas import tpu as pltpu

_NUM_CLASSES = 100
_BATCH = 4096
_NFG = _NUM_CLASSES - 1
_NPAD = 128


def _tree_nodes():
    nodes = []

    def rec(lo, hi, depth):
        if hi - lo < 2:
            return
        mid = (lo + hi) // 2
        nodes.append((lo, mid, hi, depth))
        rec(lo, mid, depth + 1)
        rec(mid, hi, depth + 1)

    rec(0, _NFG, 1)
    return nodes


_NODES = _tree_nodes()
_N_NODES = len(_NODES)


@functools.lru_cache(maxsize=None)
def _static_tables():
    sample_nums = np.arange(100, 600, 5).astype(np.float64)
    weights = (1.0 - 0.999) / (1.0 - np.power(0.999, sample_nums))
    w_fg = weights[1:]

    lo = np.zeros((_NPAD,), np.int32)
    mid = np.zeros((_NPAD,), np.int32)
    hi = np.zeros((_NPAD,), np.int32)
    wv0 = np.zeros((_NPAD,), np.float32)
    wv1 = np.zeros((_NPAD,), np.float32)
    dw = np.zeros((_NPAD,), np.float32)
    a0 = np.zeros((_NUM_CLASSES, _NPAD), np.float32)
    a1 = np.zeros((_NUM_CLASSES, _NPAD), np.float32)

    for n, (l, m, h, d) in enumerate(_NODES):
        lo[n], mid[n], hi[n] = l, m, h
        v0 = np.mean(w_fg[l:m])
        v1 = np.mean(w_fg[m:h])
        s = v0 + v1
        wv0[n] = np.float32(v0 / s * 2.0)
        wv1[n] = np.float32(v1 / s * 2.0)
        dw[n] = np.float32(d / 10.0 + 1.0)
        a0[1 + l:1 + m, n] = 1.0 / (m - l)
        a1[1 + m:1 + h, n] = 1.0 / (h - m)

    def row(x):
        return x.reshape(1, _NPAD)

    return (a0, a1, row(lo), row(mid), row(hi), row(wv0), row(wv1), row(dw))


def _body(x_ref, t_ref, a0_ref, a1_ref, lo_ref, mid_ref, hi_ref,
          w0_ref, w1_ref, dw_ref, out_ref):
    x = x_ref[...]
    t = t_ref[...]
    m0 = jnp.dot(x, a0_ref[...], preferred_element_type=jnp.float32)
    m1 = jnp.dot(x, a1_ref[...], preferred_element_type=jnp.float32)

    tf = t - 1
    fg = t != 0
    lo = lo_ref[...]
    mid = mid_ref[...]
    hi = hi_ref[...]

    mask = fg & (tf >= lo) & (tf < hi)
    tsub = tf >= mid
    d = jnp.where(tsub, m0 - m1, m1 - m0)
    nll = jnp.maximum(d, 0.0) + jnp.log1p(jnp.exp(-jnp.abs(d)))
    w = jnp.where(tsub, w1_ref[...], w0_ref[...])
    mf = mask.astype(jnp.float32)
    wm = w * mf

    num = jnp.sum(wm * nll, axis=0)
    den = jnp.sum(wm, axis=0)
    cnt = jnp.sum(mf, axis=0)

    ce = num / jnp.where(den > 0, den, 1.0)
    nonempty = (cnt > 0).astype(jnp.float32)
    loss_total = jnp.sum(nonempty * ce * dw_ref[...][0])
    node_count = jnp.sum(nonempty)
    total_samples = jnp.sum(cnt)
    n_fg = jnp.sum(fg.astype(jnp.float32))
    num_losses = n_fg * (_N_NODES / 2.0)

    out_ref[0, 0] = (loss_total / node_count) * (total_samples / num_losses)


def kernel(outputs, targets):
    (a0, a1, lo, mid, hi, wv0, wv1, dw) = _static_tables()
    t2 = targets.astype(jnp.int32).reshape(_BATCH, 1)
    out = pl.pallas_call(
        _body,
        out_shape=jax.ShapeDtypeStruct((1, 1), jnp.float32),
        out_specs=pl.BlockSpec(memory_space=pltpu.SMEM),
    )(outputs, t2, a0, a1, lo, mid, hi, wv0, wv1, dw)
    return out[0, 0]
